# Initial kernel scaffold; baseline (speedup 1.0000x reference)
import jax
import jax.numpy as jnp
from jax.experimental import pallas as pl


def kernel(coords, type_ids, type_table, W, b):
    raise NotImplementedError("write your pallas kernel here")


# speedup vs baseline: 98.2640x; 98.2640x over previous
import math

import jax
import jax.numpy as jnp
import numpy as np
from jax.experimental import pallas as pl

POS_DIM = 96
TYPE_NUMS = 10
TYPE_DIM = 96
OUT_DIM = 64
N_COORD = 6
_BLK = POS_DIM // N_COORD
_TPAD = 16

_q = np.arange(POS_DIM) % _BLK
_j = (_q // 2) * 2
_INV_FREQ = np.asarray(10000.0 ** (-2.0 * _j / POS_DIM), np.float32).reshape(1, POS_DIM)
_PHASE = np.asarray(np.where(_q % 2 == 0, 0.0, math.pi / 2.0), np.float32).reshape(1, POS_DIM)


def _enc_kernel(coords_ref, ids_ref, table_ref, w_ref, b_ref, freq_ref, phase_ref,
                out_ref):
    nrows = coords_ref.shape[0]
    sel_r = jax.lax.broadcasted_iota(jnp.int32, (N_COORD, POS_DIM), 0)
    sel_c = jax.lax.broadcasted_iota(jnp.int32, (N_COORD, POS_DIM), 1) // _BLK
    sel = (sel_r == sel_c).astype(jnp.float32)
    x = jnp.dot(coords_ref[...], sel, preferred_element_type=jnp.float32)
    pos = jnp.sin(x * freq_ref[...] + phase_ref[...])

    w1 = w_ref[:POS_DIM, :]
    w2 = w_ref[POS_DIM:, :]
    tproj = jnp.dot(table_ref[...], w2, preferred_element_type=jnp.float32)
    onehot = (ids_ref[...] == jax.lax.broadcasted_iota(
        jnp.int32, (nrows, _TPAD), 1)).astype(jnp.float32)

    acc = jnp.dot(pos, w1, preferred_element_type=jnp.float32)
    acc = acc + jnp.dot(onehot, tproj, preferred_element_type=jnp.float32)
    out_ref[...] = acc + b_ref[...]


def kernel(coords, type_ids, type_table, W, b):
    B = coords.shape[0]
    blk = 2048 if B % 2048 == 0 else B
    grid = (B // blk,)
    ids2d = type_ids.reshape(B, 1)
    table_pad = jnp.zeros((_TPAD, TYPE_DIM), jnp.float32).at[:TYPE_NUMS].set(type_table)
    b2d = b.reshape(1, OUT_DIM)
    return pl.pallas_call(
        _enc_kernel,
        grid=grid,
        in_specs=[
            pl.BlockSpec((blk, N_COORD), lambda i: (i, 0)),
            pl.BlockSpec((blk, 1), lambda i: (i, 0)),
            pl.BlockSpec((_TPAD, TYPE_DIM), lambda i: (0, 0)),
            pl.BlockSpec((TYPE_DIM + POS_DIM, OUT_DIM), lambda i: (0, 0)),
            pl.BlockSpec((1, OUT_DIM), lambda i: (0, 0)),
            pl.BlockSpec((1, POS_DIM), lambda i: (0, 0)),
            pl.BlockSpec((1, POS_DIM), lambda i: (0, 0)),
        ],
        out_specs=pl.BlockSpec((blk, OUT_DIM), lambda i: (i, 0)),
        out_shape=jax.ShapeDtypeStruct((B, OUT_DIM), jnp.float32),
    )(coords, ids2d, table_pad, W, b2d, jnp.asarray(_INV_FREQ), jnp.asarray(_PHASE))


# speedup vs baseline: 152.0025x; 1.5469x over previous
import math

import jax
import jax.numpy as jnp
import numpy as np
from jax.experimental import pallas as pl

POS_DIM = 96
TYPE_NUMS = 10
TYPE_DIM = 96
OUT_DIM = 64
N_COORD = 6
_BLK = POS_DIM // N_COORD
_TPAD = 16

_q = np.arange(POS_DIM) % _BLK
_j = (_q // 2) * 2
_INV_FREQ = np.asarray(10000.0 ** (-2.0 * _j / POS_DIM) / math.pi,
                       np.float32).reshape(1, POS_DIM)
_PHASE = np.asarray(np.where(_q % 2 == 0, 0.0, 0.5), np.float32).reshape(1, POS_DIM)

_S3 = -1.0 / 6.0
_S5 = 1.0 / 120.0
_S7 = -1.0 / 5040.0
_S9 = 1.0 / 362880.0


def _fast_sin_pi(t):
    k = jnp.round(t)
    r = (t - k) * np.float32(math.pi)
    s = r * r
    u = np.float32(_S9)
    u = u * s + np.float32(_S7)
    u = u * s + np.float32(_S5)
    u = u * s + np.float32(_S3)
    p = (r * s) * u + r
    odd = (k.astype(jnp.int32) & 1) == 1
    return jnp.where(odd, -p, p)


def _enc_kernel(coords_ref, ids_ref, table_ref, w_ref, b_ref, freq_ref, phase_ref,
                out_ref):
    nrows = coords_ref.shape[0]
    sel_r = jax.lax.broadcasted_iota(jnp.int32, (N_COORD, POS_DIM), 0)
    sel_c = jax.lax.broadcasted_iota(jnp.int32, (N_COORD, POS_DIM), 1) // _BLK
    sel = (sel_r == sel_c).astype(jnp.float32)
    x = jnp.dot(coords_ref[...], sel, preferred_element_type=jnp.float32)
    pos = _fast_sin_pi(x * freq_ref[...] + phase_ref[...])

    w1 = w_ref[:POS_DIM, :]
    w2 = w_ref[POS_DIM:, :]
    tproj = jnp.dot(table_ref[...], w2, preferred_element_type=jnp.float32)
    onehot = (ids_ref[...] == jax.lax.broadcasted_iota(
        jnp.int32, (nrows, _TPAD), 1)).astype(jnp.float32)

    acc = jnp.dot(pos, w1, preferred_element_type=jnp.float32)
    acc = acc + jnp.dot(onehot, tproj, preferred_element_type=jnp.float32)
    out_ref[...] = acc + b_ref[...]


def kernel(coords, type_ids, type_table, W, b):
    B = coords.shape[0]
    blk = 2048 if B % 2048 == 0 else B
    grid = (B // blk,)
    ids2d = type_ids.reshape(B, 1)
    table_pad = jnp.zeros((_TPAD, TYPE_DIM), jnp.float32).at[:TYPE_NUMS].set(type_table)
    b2d = b.reshape(1, OUT_DIM)
    return pl.pallas_call(
        _enc_kernel,
        grid=grid,
        in_specs=[
            pl.BlockSpec((blk, N_COORD), lambda i: (i, 0)),
            pl.BlockSpec((blk, 1), lambda i: (i, 0)),
            pl.BlockSpec((_TPAD, TYPE_DIM), lambda i: (0, 0)),
            pl.BlockSpec((TYPE_DIM + POS_DIM, OUT_DIM), lambda i: (0, 0)),
            pl.BlockSpec((1, OUT_DIM), lambda i: (0, 0)),
            pl.BlockSpec((1, POS_DIM), lambda i: (0, 0)),
            pl.BlockSpec((1, POS_DIM), lambda i: (0, 0)),
        ],
        out_specs=pl.BlockSpec((blk, OUT_DIM), lambda i: (i, 0)),
        out_shape=jax.ShapeDtypeStruct((B, OUT_DIM), jnp.float32),
    )(coords, ids2d, table_pad, W, b2d, jnp.asarray(_INV_FREQ), jnp.asarray(_PHASE))


# speedup vs baseline: 161.2072x; 1.0606x over previous
import math

import jax
import jax.numpy as jnp
import numpy as np
from jax.experimental import pallas as pl

POS_DIM = 96
TYPE_NUMS = 10
TYPE_DIM = 96
OUT_DIM = 64
N_COORD = 6
_BLK = POS_DIM // N_COORD
_TPAD = 16

_q = np.arange(POS_DIM) % _BLK
_j = (_q // 2) * 2
_INV_FREQ = np.asarray(10000.0 ** (-2.0 * _j / POS_DIM) / math.pi,
                       np.float32).reshape(1, POS_DIM)
_PHASE = np.asarray(np.where(_q % 2 == 0, 0.0, 0.5), np.float32).reshape(1, POS_DIM)

_S3 = -1.0 / 6.0
_S5 = 1.0 / 120.0
_S7 = -1.0 / 5040.0
_S9 = 1.0 / 362880.0


def _fast_sin_pi(t):
    k = jnp.round(t)
    r = (t - k) * np.float32(math.pi)
    s = r * r
    u = np.float32(_S9)
    u = u * s + np.float32(_S7)
    u = u * s + np.float32(_S5)
    u = u * s + np.float32(_S3)
    p = (r * s) * u + r
    odd = (k.astype(jnp.int32) & 1) == 1
    return jnp.where(odd, -p, p)


def _enc_kernel(coords_ref, ids_ref, table_ref, w_ref, b_ref, freq_ref, phase_ref,
                out_ref):
    nrows = coords_ref.shape[0]
    sel_r = jax.lax.broadcasted_iota(jnp.int32, (N_COORD, POS_DIM), 0)
    sel_c = jax.lax.broadcasted_iota(jnp.int32, (N_COORD, POS_DIM), 1) // _BLK
    sel = (sel_r == sel_c).astype(jnp.float32)
    x = jnp.dot(coords_ref[...], sel, preferred_element_type=jnp.float32)
    pos = _fast_sin_pi(x * freq_ref[...] + phase_ref[...])

    w1 = w_ref[:POS_DIM, :]
    w2 = w_ref[POS_DIM:, :]
    tproj = jnp.dot(table_ref[...], w2, preferred_element_type=jnp.float32)
    onehot = (ids_ref[...] == jax.lax.broadcasted_iota(
        jnp.int32, (nrows, _TPAD), 1)).astype(jnp.float32)

    acc = jnp.dot(pos, w1, preferred_element_type=jnp.float32)
    acc = acc + jnp.dot(onehot, tproj, preferred_element_type=jnp.float32)
    out_ref[...] = acc + b_ref[...]


def kernel(coords, type_ids, type_table, W, b):
    B = coords.shape[0]
    blk = 4096 if B % 4096 == 0 else B
    grid = (B // blk,)
    ids2d = type_ids.reshape(B, 1)
    table_pad = jnp.zeros((_TPAD, TYPE_DIM), jnp.float32).at[:TYPE_NUMS].set(type_table)
    b2d = b.reshape(1, OUT_DIM)
    return pl.pallas_call(
        _enc_kernel,
        grid=grid,
        in_specs=[
            pl.BlockSpec((blk, N_COORD), lambda i: (i, 0)),
            pl.BlockSpec((blk, 1), lambda i: (i, 0)),
            pl.BlockSpec((_TPAD, TYPE_DIM), lambda i: (0, 0)),
            pl.BlockSpec((TYPE_DIM + POS_DIM, OUT_DIM), lambda i: (0, 0)),
            pl.BlockSpec((1, OUT_DIM), lambda i: (0, 0)),
            pl.BlockSpec((1, POS_DIM), lambda i: (0, 0)),
            pl.BlockSpec((1, POS_DIM), lambda i: (0, 0)),
        ],
        out_specs=pl.BlockSpec((blk, OUT_DIM), lambda i: (i, 0)),
        out_shape=jax.ShapeDtypeStruct((B, OUT_DIM), jnp.float32),
    )(coords, ids2d, table_pad, W, b2d, jnp.asarray(_INV_FREQ), jnp.asarray(_PHASE))


# speedup vs baseline: 161.2807x; 1.0005x over previous
import math

import jax
import jax.numpy as jnp
import numpy as np
from jax.experimental import pallas as pl
from jax.experimental.pallas import tpu as pltpu

POS_DIM = 96
TYPE_NUMS = 10
TYPE_DIM = 96
OUT_DIM = 64
N_COORD = 6
_BLK = POS_DIM // N_COORD
_TPAD = 16

_q = np.arange(POS_DIM) % _BLK
_j = (_q // 2) * 2
_INV_FREQ = np.asarray(10000.0 ** (-2.0 * _j / POS_DIM) / math.pi,
                       np.float32).reshape(1, POS_DIM)
_PHASE = np.asarray(np.where(_q % 2 == 0, 0.0, 0.5), np.float32).reshape(1, POS_DIM)

_S3 = -1.0 / 6.0
_S5 = 1.0 / 120.0
_S7 = -1.0 / 5040.0
_S9 = 1.0 / 362880.0


def _fast_sin_pi(t):
    k = jnp.round(t)
    r = (t - k) * np.float32(math.pi)
    s = r * r
    u = np.float32(_S9)
    u = u * s + np.float32(_S7)
    u = u * s + np.float32(_S5)
    u = u * s + np.float32(_S3)
    p = (r * s) * u + r
    odd = (k.astype(jnp.int32) & 1) == 1
    return jnp.where(odd, -p, p)


def _enc_kernel(coords_ref, ids_ref, table_ref, w_ref, b_ref, freq_ref, phase_ref,
                out_ref):
    nrows = coords_ref.shape[0]
    sel_r = jax.lax.broadcasted_iota(jnp.int32, (N_COORD, POS_DIM), 0)
    sel_c = jax.lax.broadcasted_iota(jnp.int32, (N_COORD, POS_DIM), 1) // _BLK
    sel = (sel_r == sel_c).astype(jnp.float32)
    x = jnp.dot(coords_ref[...], sel, preferred_element_type=jnp.float32)
    pos = _fast_sin_pi(x * freq_ref[...] + phase_ref[...])

    w1 = w_ref[:POS_DIM, :]
    w2 = w_ref[POS_DIM:, :]
    tproj = jnp.dot(table_ref[...], w2, preferred_element_type=jnp.float32)
    onehot = (ids_ref[...] == jax.lax.broadcasted_iota(
        jnp.int32, (nrows, _TPAD), 1)).astype(jnp.float32)

    acc = jnp.dot(pos, w1, preferred_element_type=jnp.float32)
    acc = acc + jnp.dot(onehot, tproj, preferred_element_type=jnp.float32)
    out_ref[...] = acc + b_ref[...]


def kernel(coords, type_ids, type_table, W, b):
    B = coords.shape[0]
    blk = 4096 if B % 4096 == 0 else B
    grid = (B // blk,)
    ids2d = type_ids.reshape(B, 1)
    table_pad = jnp.zeros((_TPAD, TYPE_DIM), jnp.float32).at[:TYPE_NUMS].set(type_table)
    b2d = b.reshape(1, OUT_DIM)
    return pl.pallas_call(
        _enc_kernel,
        grid=grid,
        in_specs=[
            pl.BlockSpec((blk, N_COORD), lambda i: (i, 0)),
            pl.BlockSpec((blk, 1), lambda i: (i, 0)),
            pl.BlockSpec((_TPAD, TYPE_DIM), lambda i: (0, 0)),
            pl.BlockSpec((TYPE_DIM + POS_DIM, OUT_DIM), lambda i: (0, 0)),
            pl.BlockSpec((1, OUT_DIM), lambda i: (0, 0)),
            pl.BlockSpec((1, POS_DIM), lambda i: (0, 0)),
            pl.BlockSpec((1, POS_DIM), lambda i: (0, 0)),
        ],
        out_specs=pl.BlockSpec((blk, OUT_DIM), lambda i: (i, 0)),
        out_shape=jax.ShapeDtypeStruct((B, OUT_DIM), jnp.float32),
        compiler_params=pltpu.CompilerParams(dimension_semantics=("parallel",)),
    )(coords, ids2d, table_pad, W, b2d, jnp.asarray(_INV_FREQ), jnp.asarray(_PHASE))


# speedup vs baseline: 164.9366x; 1.0227x over previous
import math

import jax
import jax.numpy as jnp
import numpy as np
from jax.experimental import pallas as pl
from jax.experimental.pallas import tpu as pltpu

POS_DIM = 96
TYPE_NUMS = 10
TYPE_DIM = 96
OUT_DIM = 64
N_COORD = 6
_BLK = POS_DIM // N_COORD
_TPAD = 16

_q = np.arange(POS_DIM) % _BLK
_j = (_q // 2) * 2
_INV_FREQ = np.asarray(10000.0 ** (-2.0 * _j / POS_DIM) / math.pi,
                       np.float32).reshape(1, POS_DIM)
_PHASE = np.asarray(np.where(_q % 2 == 0, 0.0, 0.5), np.float32).reshape(1, POS_DIM)

_S3 = -0.16665682437313756
_S5 = 0.008312384897448729
_S7 = -0.00018492760381430236


def _fast_sin_pi(t):
    k = jnp.round(t)
    r = (t - k) * np.float32(math.pi)
    s = r * r
    u = np.float32(_S7)
    u = u * s + np.float32(_S5)
    u = u * s + np.float32(_S3)
    p = (r * s) * u + r
    flip = jax.lax.shift_left(k.astype(jnp.int32), 31)
    bits = jax.lax.bitcast_convert_type(p, jnp.int32) ^ flip
    return jax.lax.bitcast_convert_type(bits, jnp.float32)


def _enc_kernel(coords_ref, ids_ref, table_ref, w_ref, b_ref, freq_ref, phase_ref,
                out_ref):
    nrows = coords_ref.shape[0]
    sel_r = jax.lax.broadcasted_iota(jnp.int32, (N_COORD, POS_DIM), 0)
    sel_c = jax.lax.broadcasted_iota(jnp.int32, (N_COORD, POS_DIM), 1) // _BLK
    sel = (sel_r == sel_c).astype(jnp.float32)
    x = jnp.dot(coords_ref[...], sel, preferred_element_type=jnp.float32)
    pos = _fast_sin_pi(x * freq_ref[...] + phase_ref[...])

    w1 = w_ref[:POS_DIM, :]
    w2 = w_ref[POS_DIM:, :]
    tproj = jnp.dot(table_ref[...], w2, preferred_element_type=jnp.float32)
    onehot = (ids_ref[...] == jax.lax.broadcasted_iota(
        jnp.int32, (nrows, _TPAD), 1)).astype(jnp.float32)

    acc = jnp.dot(pos, w1, preferred_element_type=jnp.float32)
    acc = acc + jnp.dot(onehot, tproj, preferred_element_type=jnp.float32)
    out_ref[...] = acc + b_ref[...]


def kernel(coords, type_ids, type_table, W, b):
    B = coords.shape[0]
    blk = 4096 if B % 4096 == 0 else B
    grid = (B // blk,)
    ids2d = type_ids.reshape(B, 1)
    table_pad = jnp.zeros((_TPAD, TYPE_DIM), jnp.float32).at[:TYPE_NUMS].set(type_table)
    b2d = b.reshape(1, OUT_DIM)
    return pl.pallas_call(
        _enc_kernel,
        grid=grid,
        in_specs=[
            pl.BlockSpec((blk, N_COORD), lambda i: (i, 0)),
            pl.BlockSpec((blk, 1), lambda i: (i, 0)),
            pl.BlockSpec((_TPAD, TYPE_DIM), lambda i: (0, 0)),
            pl.BlockSpec((TYPE_DIM + POS_DIM, OUT_DIM), lambda i: (0, 0)),
            pl.BlockSpec((1, OUT_DIM), lambda i: (0, 0)),
            pl.BlockSpec((1, POS_DIM), lambda i: (0, 0)),
            pl.BlockSpec((1, POS_DIM), lambda i: (0, 0)),
        ],
        out_specs=pl.BlockSpec((blk, OUT_DIM), lambda i: (i, 0)),
        out_shape=jax.ShapeDtypeStruct((B, OUT_DIM), jnp.float32),
        compiler_params=pltpu.CompilerParams(dimension_semantics=("parallel",)),
    )(coords, ids2d, table_pad, W, b2d, jnp.asarray(_INV_FREQ), jnp.asarray(_PHASE))


# speedup vs baseline: 245.1268x; 1.4862x over previous
import math

import jax
import jax.numpy as jnp
import numpy as np
from jax.experimental import pallas as pl
from jax.experimental.pallas import tpu as pltpu

POS_DIM = 96
TYPE_NUMS = 10
TYPE_DIM = 96
OUT_DIM = 64
N_COORD = 6
_BLK = POS_DIM // N_COORD
_TPAD = 16
_KIN = 8

_p = np.arange(POS_DIM)
_q = _p % _BLK
_j = (_q // 2) * 2
_freq = 10000.0 ** (-2.0 * _j / POS_DIM) / math.pi
_phase = np.where(_q % 2 == 0, 0.0, 0.5)
_selT = np.zeros((POS_DIM + _TPAD, _KIN), np.float32)
_selT[_p, _p // _BLK] = _freq
_selT[_p, 6] = _phase
_selT[POS_DIM + np.arange(_TPAD), 7] = 1.0
_selT[POS_DIM + np.arange(_TPAD), 6] = -np.arange(_TPAD)

_S3 = -0.16665682437313756
_S5 = 0.008312384897448729
_S7 = -0.00018492760381430236


def _fast_sin_pi(t):
    k = jnp.round(t)
    r = (t - k) * np.float32(math.pi)
    s = r * r
    u = np.float32(_S7)
    u = u * s + np.float32(_S5)
    u = u * s + np.float32(_S3)
    p = (r * s) * u + r
    flip = jax.lax.shift_left(k.astype(jnp.int32), 31)
    bits = jax.lax.bitcast_convert_type(p, jnp.int32) ^ flip
    return jax.lax.bitcast_convert_type(bits, jnp.float32)


_TDIMS = (((0,), (0,)), ((), ()))


def _enc_kernel(zin_ref, selt_ref, table_ref, w_ref, b_ref, out_ref):
    at = jnp.dot(selt_ref[...], zin_ref[...], preferred_element_type=jnp.float32)
    pos_t = _fast_sin_pi(at[:POS_DIM, :])
    oh_t = jnp.maximum(1.0 - jnp.abs(at[POS_DIM:, :]), 0.0)

    w1 = w_ref[:POS_DIM, :]
    w2 = w_ref[POS_DIM:, :]
    tproj = jnp.dot(table_ref[...], w2, preferred_element_type=jnp.float32)

    acc = jax.lax.dot_general(pos_t, w1, _TDIMS, preferred_element_type=jnp.float32)
    acc = acc + jax.lax.dot_general(oh_t, tproj, _TDIMS,
                                    preferred_element_type=jnp.float32)
    out_ref[...] = acc + b_ref[...]


def kernel(coords, type_ids, type_table, W, b):
    B = coords.shape[0]
    blk = 4096 if B % 4096 == 0 else B
    grid = (B // blk,)
    zin = jnp.concatenate(
        [coords.T, jnp.ones((1, B), jnp.float32),
         type_ids.astype(jnp.float32).reshape(1, B)], axis=0)
    table_pad = jnp.zeros((_TPAD, TYPE_DIM), jnp.float32).at[:TYPE_NUMS].set(type_table)
    b2d = b.reshape(1, OUT_DIM)
    return pl.pallas_call(
        _enc_kernel,
        grid=grid,
        in_specs=[
            pl.BlockSpec((_KIN, blk), lambda i: (0, i)),
            pl.BlockSpec((POS_DIM + _TPAD, _KIN), lambda i: (0, 0)),
            pl.BlockSpec((_TPAD, TYPE_DIM), lambda i: (0, 0)),
            pl.BlockSpec((TYPE_DIM + POS_DIM, OUT_DIM), lambda i: (0, 0)),
            pl.BlockSpec((1, OUT_DIM), lambda i: (0, 0)),
        ],
        out_specs=pl.BlockSpec((blk, OUT_DIM), lambda i: (i, 0)),
        out_shape=jax.ShapeDtypeStruct((B, OUT_DIM), jnp.float32),
        compiler_params=pltpu.CompilerParams(dimension_semantics=("parallel",)),
    )(zin, jnp.asarray(_selT), table_pad, W, b2d)
